# Initial kernel scaffold; baseline (speedup 1.0000x reference)
import jax
import jax.numpy as jnp
from jax.experimental import pallas as pl


def kernel(indices, mask, embedding, kernel):
    raise NotImplementedError("write your pallas kernel here")


# speedup vs baseline: 1.1130x; 1.1130x over previous
import functools

import jax
import jax.numpy as jnp
from jax import lax
from jax.experimental import pallas as pl
from jax.experimental.pallas import tpu as pltpu
from jax.experimental.pallas import tpu_sc as plsc

B = 1024
LSEQ = 200
D = 64
NLBL = 100000

NC = 2
NS = 16
NW = NC * NS
ROWS_PER_W = B // NW

def _sc_body(idx_hbm, mask_hbm, emb_hbm, out_hbm,
             idx_v, mask_v, rows_v, sums_v, sem):
    wid = lax.axis_index("s") * NC + lax.axis_index("c")
    base = wid * ROWS_PER_W
    pltpu.sync_copy(idx_hbm.at[pl.ds(base, ROWS_PER_W)], idx_v)
    pltpu.sync_copy(mask_hbm.at[pl.ds(base, ROWS_PER_W)], mask_v)
    for b in range(ROWS_PER_W):
        pltpu.async_copy(emb_hbm.at[idx_v.at[b]], rows_v, sem).wait()

        def body(l, accs, b=b):
            m = plsc.load_gather(
                mask_v,
                [jnp.full((16,), b, jnp.int32), jnp.full((16,), l, jnp.int32)],
            )
            return tuple(
                accs[j] + rows_v[l, pl.ds(j * 16, 16)] * m for j in range(4)
            )

        accs = lax.fori_loop(
            0, LSEQ, body,
            tuple(jnp.zeros((16,), jnp.float32) for _ in range(4)),
        )
        for j in range(4):
            sums_v[b, pl.ds(j * 16, 16)] = accs[j]
    pltpu.sync_copy(sums_v, out_hbm.at[pl.ds(base, ROWS_PER_W)])


@functools.cache
def _sc_masked_sum_fn():
    mesh = plsc.VectorSubcoreMesh(
        core_axis_name="c", subcore_axis_name="s", num_cores=NC, num_subcores=NS
    )
    return pl.kernel(
        _sc_body,
        out_type=jax.ShapeDtypeStruct((B, D), jnp.float32),
        mesh=mesh,
        scratch_types=[
            pltpu.VMEM((ROWS_PER_W, LSEQ), jnp.int32),
            pltpu.VMEM((ROWS_PER_W, LSEQ), jnp.float32),
            pltpu.VMEM((LSEQ, D), jnp.float32),
            pltpu.VMEM((ROWS_PER_W, D), jnp.float32),
            pltpu.SemaphoreType.DMA,
        ],
        compiler_params=pltpu.CompilerParams(
            use_tc_tiling_on_sc=False, needs_layout_passes=False
        ),
    )


BLK_N = 2048
_GRID_N = (NLBL + BLK_N - 1) // BLK_N


def _tc_body(sums_ref, mask_ref, w_ref, out_ref, normed_ref):
    @pl.when(pl.program_id(0) == 0)
    def _():
        msum = jnp.sum(mask_ref[...], axis=1, keepdims=True)
        v = sums_ref[...] / jnp.maximum(msum, 1.0)
        nrm = jnp.sqrt(jnp.sum(v * v, axis=1, keepdims=True))
        normed_ref[...] = v / jnp.maximum(nrm, 1e-4)

    out_ref[...] = jnp.dot(
        normed_ref[...], w_ref[...],
        preferred_element_type=jnp.float32,
        precision=lax.Precision.HIGHEST,
    )


def _tc_matmul(sums, mask, w):
    return pl.pallas_call(
        _tc_body,
        grid=(_GRID_N,),
        in_specs=[
            pl.BlockSpec((B, D), lambda i: (0, 0)),
            pl.BlockSpec((B, LSEQ), lambda i: (0, 0)),
            pl.BlockSpec((D, BLK_N), lambda i: (0, i)),
        ],
        out_specs=pl.BlockSpec((B, BLK_N), lambda i: (0, i)),
        out_shape=jax.ShapeDtypeStruct((B, NLBL), jnp.float32),
        scratch_shapes=[pltpu.VMEM((B, D), jnp.float32)],
    )(sums, mask, w)


def kernel(indices, mask, embedding, kernel):
    sums = _sc_masked_sum_fn()(indices, mask, embedding)
    return _tc_matmul(sums, mask, kernel)


# speedup vs baseline: 1.4594x; 1.3113x over previous
import functools

import jax
import jax.numpy as jnp
from jax import lax
from jax.experimental import pallas as pl
from jax.experimental.pallas import tpu as pltpu
from jax.experimental.pallas import tpu_sc as plsc

B = 1024
LSEQ = 200
D = 64
NLBL = 100000

NC = 2
NS = 16
NW = NC * NS
ROWS_PER_W = B // NW

def _sc_body(idx_hbm, mask_hbm, emb_hbm, out_hbm,
             idx_v, mask_v, rows_v, sums_v, sem):
    wid = lax.axis_index("s") * NC + lax.axis_index("c")
    base = wid * ROWS_PER_W
    pltpu.sync_copy(idx_hbm.at[pl.ds(base, ROWS_PER_W)], idx_v)
    pltpu.sync_copy(mask_hbm.at[pl.ds(base, ROWS_PER_W)], mask_v)
    for b in range(ROWS_PER_W):
        pltpu.async_copy(emb_hbm.at[idx_v.at[b]], rows_v, sem).wait()

        def body(l, accs, b=b):
            m = plsc.load_gather(
                mask_v,
                [jnp.full((16,), b, jnp.int32), jnp.full((16,), l, jnp.int32)],
            )
            return tuple(
                accs[j] + rows_v[l, pl.ds(j * 16, 16)] * m for j in range(4)
            )

        accs = lax.fori_loop(
            0, LSEQ, body,
            tuple(jnp.zeros((16,), jnp.float32) for _ in range(4)),
        )
        for j in range(4):
            sums_v[b, pl.ds(j * 16, 16)] = accs[j]
    pltpu.sync_copy(sums_v, out_hbm.at[pl.ds(base, ROWS_PER_W)])


@functools.cache
def _sc_masked_sum_fn():
    mesh = plsc.VectorSubcoreMesh(
        core_axis_name="c", subcore_axis_name="s", num_cores=NC, num_subcores=NS
    )
    return pl.kernel(
        _sc_body,
        out_type=jax.ShapeDtypeStruct((B, D), jnp.float32),
        mesh=mesh,
        scratch_types=[
            pltpu.VMEM((ROWS_PER_W, LSEQ), jnp.int32),
            pltpu.VMEM((ROWS_PER_W, LSEQ), jnp.float32),
            pltpu.VMEM((LSEQ, D), jnp.float32),
            pltpu.VMEM((ROWS_PER_W, D), jnp.float32),
            pltpu.SemaphoreType.DMA,
        ],
        compiler_params=pltpu.CompilerParams(
            use_tc_tiling_on_sc=False, needs_layout_passes=False
        ),
    )


BLK_N = 2048
_GRID_N = (NLBL + BLK_N - 1) // BLK_N


def _tc_body(sums_ref, mask_ref, w_ref, out_ref, normed_ref):
    @pl.when(pl.program_id(0) == 0)
    def _():
        msum = jnp.sum(mask_ref[...], axis=1, keepdims=True)
        v = sums_ref[...] / jnp.maximum(msum, 1.0)
        nrm = jnp.sqrt(jnp.sum(v * v, axis=1, keepdims=True))
        normed_ref[...] = v / jnp.maximum(nrm, 1e-4)

    out_ref[...] = jnp.dot(
        normed_ref[...], w_ref[...],
        preferred_element_type=jnp.float32,
        precision=lax.Precision.DEFAULT,
    )


def _tc_matmul(sums, mask, w):
    return pl.pallas_call(
        _tc_body,
        grid=(_GRID_N,),
        in_specs=[
            pl.BlockSpec((B, D), lambda i: (0, 0)),
            pl.BlockSpec((B, LSEQ), lambda i: (0, 0)),
            pl.BlockSpec((D, BLK_N), lambda i: (0, i)),
        ],
        out_specs=pl.BlockSpec((B, BLK_N), lambda i: (0, i)),
        out_shape=jax.ShapeDtypeStruct((B, NLBL), jnp.float32),
        scratch_shapes=[pltpu.VMEM((B, D), jnp.float32)],
    )(sums, mask, w)


def kernel(indices, mask, embedding, kernel):
    sums = _sc_masked_sum_fn()(indices, mask, embedding)
    return _tc_matmul(sums, mask, kernel)


# speedup vs baseline: 1.4658x; 1.0044x over previous
import functools

import jax
import jax.numpy as jnp
from jax import lax
from jax.experimental import pallas as pl
from jax.experimental.pallas import tpu as pltpu
from jax.experimental.pallas import tpu_sc as plsc

B = 1024
LSEQ = 200
D = 64
NLBL = 100000

NC = 2
NS = 16
NW = NC * NS
ROWS_PER_W = B // NW

def _sc_body(idx_hbm, mask_hbm, emb_hbm, out_hbm,
             idx_v, mask_v, rows_v, sums_v, sem):
    wid = lax.axis_index("s") * NC + lax.axis_index("c")
    base = wid * ROWS_PER_W
    pltpu.sync_copy(idx_hbm.at[pl.ds(base, ROWS_PER_W)], idx_v)
    pltpu.sync_copy(mask_hbm.at[pl.ds(base, ROWS_PER_W)], mask_v)
    for b in range(ROWS_PER_W):
        pltpu.async_copy(emb_hbm.at[idx_v.at[b]], rows_v, sem).wait()

        def body(l, accs, b=b):
            m = plsc.load_gather(
                mask_v,
                [jnp.full((16,), b, jnp.int32), jnp.full((16,), l, jnp.int32)],
            )
            return tuple(
                accs[j] + rows_v[l, pl.ds(j * 16, 16)] * m for j in range(4)
            )

        accs = lax.fori_loop(
            0, LSEQ, body,
            tuple(jnp.zeros((16,), jnp.float32) for _ in range(4)),
        )
        for j in range(4):
            sums_v[b, pl.ds(j * 16, 16)] = accs[j]
    pltpu.sync_copy(sums_v, out_hbm.at[pl.ds(base, ROWS_PER_W)])


@functools.cache
def _sc_masked_sum_fn():
    mesh = plsc.VectorSubcoreMesh(
        core_axis_name="c", subcore_axis_name="s", num_cores=NC, num_subcores=NS
    )
    return pl.kernel(
        _sc_body,
        out_type=jax.ShapeDtypeStruct((B, D), jnp.float32),
        mesh=mesh,
        scratch_types=[
            pltpu.VMEM((ROWS_PER_W, LSEQ), jnp.int32),
            pltpu.VMEM((ROWS_PER_W, LSEQ), jnp.float32),
            pltpu.VMEM((LSEQ, D), jnp.float32),
            pltpu.VMEM((ROWS_PER_W, D), jnp.float32),
            pltpu.SemaphoreType.DMA,
        ],
        compiler_params=pltpu.CompilerParams(
            use_tc_tiling_on_sc=False, needs_layout_passes=False
        ),
    )


BLK_N = 4096
_GRID_N = (NLBL + BLK_N - 1) // BLK_N


def _tc_body(sums_ref, mask_ref, w_ref, out_ref, normed_ref):
    @pl.when(pl.program_id(0) == 0)
    def _():
        msum = jnp.sum(mask_ref[...], axis=1, keepdims=True)
        v = sums_ref[...] / jnp.maximum(msum, 1.0)
        nrm = jnp.sqrt(jnp.sum(v * v, axis=1, keepdims=True))
        normed_ref[...] = v / jnp.maximum(nrm, 1e-4)

    out_ref[...] = jnp.dot(
        normed_ref[...], w_ref[...],
        preferred_element_type=jnp.float32,
        precision=lax.Precision.DEFAULT,
    )


def _tc_matmul(sums, mask, w):
    return pl.pallas_call(
        _tc_body,
        grid=(_GRID_N,),
        in_specs=[
            pl.BlockSpec((B, D), lambda i: (0, 0)),
            pl.BlockSpec((B, LSEQ), lambda i: (0, 0)),
            pl.BlockSpec((D, BLK_N), lambda i: (0, i)),
        ],
        out_specs=pl.BlockSpec((B, BLK_N), lambda i: (0, i)),
        out_shape=jax.ShapeDtypeStruct((B, NLBL), jnp.float32),
        scratch_shapes=[pltpu.VMEM((B, D), jnp.float32)],
    )(sums, mask, w)


def kernel(indices, mask, embedding, kernel):
    sums = _sc_masked_sum_fn()(indices, mask, embedding)
    return _tc_matmul(sums, mask, kernel)
